# baseline (speedup 1.0000x reference)
import functools

import jax
import jax.numpy as jnp
from jax import lax
from jax.experimental import pallas as pl
from jax.experimental.pallas import tpu as pltpu
from jax.experimental.pallas import tpu_sc as plsc

_INFO = plsc.get_sparse_core_info()
_NC = _INFO.num_cores
_NS = _INFO.num_subcores
_NW = _NC * _NS

_B = 1024
_S = 200
_D = 128
_EPS = 1e-5

_TOK = _B * _S
_PER_W = _TOK // _NW
_CHUNK = 128
_NCHUNK = _PER_W // _CHUNK
_V8 = _D // 16


def _rsqrt(x):
    i = lax.bitcast_convert_type(x, jnp.int32)
    i = jnp.int32(0x5F3759DF) - lax.shift_right_logical(i, 1)
    y = lax.bitcast_convert_type(i, jnp.float32)
    for _ in range(3):
        y = y * (1.5 - 0.5 * x * y * y)
    return y


def _body(ids_hbm, word_hbm, pos_hbm, gamma_hbm, beta_hbm, out_hbm,
          idx_v, pos_v, g_v, b_v, r0, r1, o0, o1, sg0, sg1, ss0, ss1):
    cid = lax.axis_index("c")
    sid = lax.axis_index("s")
    wid = sid * _NC + cid

    pltpu.sync_copy(ids_hbm.at[wid], idx_v)
    pltpu.sync_copy(pos_hbm.at[pl.ds(0, _S)], pos_v)
    pltpu.sync_copy(gamma_hbm, g_v)
    pltpu.sync_copy(beta_hbm, b_v)

    pltpu.async_copy(word_hbm.at[idx_v.at[0]], r0, sg0)
    pltpu.async_copy(word_hbm.at[idx_v.at[1]], r1, sg1)

    def compute(c, r, o):
        def grp(g, carry):
            ids16 = idx_v[c, pl.ds(g * 16, 16)]
            for j16 in range(16):
                j = g * 16 + j16
                id_s = ids16[j16]
                m = jnp.where(id_s == 0, jnp.float32(0.0), jnp.float32(1.0))
                p = lax.rem(c * _CHUNK + j, _S)
                x = [r[j, pl.ds(16 * v, 16)] * m + pos_v[p, pl.ds(16 * v, 16)]
                     for v in range(_V8)]
                s01 = (x[0] + x[1]) + (x[2] + x[3])
                s23 = (x[4] + x[5]) + (x[6] + x[7])
                mean = jnp.sum(s01 + s23) * (1.0 / _D)
                xc = [xv - mean for xv in x]
                q = [xv * xv for xv in xc]
                q01 = (q[0] + q[1]) + (q[2] + q[3])
                q23 = (q[4] + q[5]) + (q[6] + q[7])
                var = jnp.sum(q01 + q23) * (1.0 / _D)
                rs = _rsqrt(var + _EPS)
                for v in range(_V8):
                    o[j, pl.ds(16 * v, 16)] = (
                        xc[v] * rs * g_v[pl.ds(16 * v, 16)]
                        + b_v[pl.ds(16 * v, 16)])
            return carry
        lax.fori_loop(0, _CHUNK // 16, grp, None)

    def outer(i, carry):
        c0 = 2 * i
        for b, (r, o, sg, ss) in enumerate(
                ((r0, o0, sg0, ss0), (r1, o1, sg1, ss1))):
            c = c0 + b
            pltpu.make_async_copy(word_hbm.at[pl.ds(0, _CHUNK)], r, sg).wait()

            @pl.when(c >= 2)
            def _():
                pltpu.make_async_copy(o, out_hbm.at[0], ss).wait()

            compute(c, r, o)
            pltpu.async_copy(o, out_hbm.at[wid * _NCHUNK + c], ss)

            @pl.when(c + 2 < _NCHUNK)
            def _():
                pltpu.async_copy(word_hbm.at[idx_v.at[c + 2]], r, sg)
        return carry

    lax.fori_loop(0, _NCHUNK // 2, outer, None)

    pltpu.make_async_copy(o0, out_hbm.at[0], ss0).wait()
    pltpu.make_async_copy(o1, out_hbm.at[0], ss1).wait()


_emb_ln = pl.kernel(
    _body,
    out_type=jax.ShapeDtypeStruct((_NW * _NCHUNK, _CHUNK, _D), jnp.float32),
    mesh=plsc.VectorSubcoreMesh(core_axis_name="c", subcore_axis_name="s"),
    compiler_params=pltpu.CompilerParams(needs_layout_passes=False),
    scratch_types=[
        pltpu.VMEM((_NCHUNK, _CHUNK), jnp.int32),
        pltpu.VMEM((_S, _D), jnp.float32),
        pltpu.VMEM((_D,), jnp.float32),
        pltpu.VMEM((_D,), jnp.float32),
        pltpu.VMEM((_CHUNK, _D), jnp.float32),
        pltpu.VMEM((_CHUNK, _D), jnp.float32),
        pltpu.VMEM((_CHUNK, _D), jnp.float32),
        pltpu.VMEM((_CHUNK, _D), jnp.float32),
        pltpu.SemaphoreType.DMA,
        pltpu.SemaphoreType.DMA,
        pltpu.SemaphoreType.DMA,
        pltpu.SemaphoreType.DMA,
    ],
)


def kernel(input_ids, word_table, pos_table, gamma, beta):
    B, S = input_ids.shape
    ids3 = input_ids.astype(jnp.int32).reshape(_NW, _NCHUNK, _CHUNK)
    out = _emb_ln(ids3, word_table, pos_table, gamma, beta)
    return out.reshape(B, S, _D)
